# baseline (speedup 1.0000x reference)
import jax
import jax.numpy as jnp
import numpy as np
from jax import lax
from jax.experimental import pallas as pl
from jax.experimental.pallas import tpu as pltpu
from jax.experimental.pallas import tpu_sc as plsc

VOCAB = 100000
EMBED = 128
BATCH = 4096
HIST = 200

NWORKERS = 32
BPW = BATCH // NWORKERS
LOMASK = np.int32(0xFFFF)
HIMASK = np.int32(-65536)
NFULL = HIST // 16
NTAIL = HIST - NFULL * 16


def _body(idx_hbm, par_hbm, weight_hbm, out_hbm, idx_v, par_v, rows_v,
          out_v, sem):
    wid = lax.axis_index("s") * 2 + lax.axis_index("c")
    base_b = wid * BPW
    weight_i32 = weight_hbm.bitcast(jnp.int32)

    iota = lax.iota(jnp.int32, 16)
    l_idx = [jnp.minimum(lc * 16 + iota, HIST - 1) for lc in range(NFULL + 1)]
    tail_mask = iota < NTAIL

    def per_batch(bi, _):
        b = base_b + bi
        pltpu.sync_copy(idx_hbm.at[b], idx_v)
        pltpu.sync_copy(par_hbm.at[b], par_v)
        g0 = pltpu.async_copy(
            weight_i32.at[idx_v.at[0]], rows_v.at[pl.ds(0, 100)], sem)
        g1 = pltpu.async_copy(
            weight_i32.at[idx_v.at[1]], rows_v.at[pl.ds(100, 100)], sem)
        g0.wait()
        g1.wait()

        sh_lo = [plsc.load_gather(par_v, [l_idx[lc]])
                 for lc in range(NFULL + 1)]
        sh_hi = [16 - s for s in sh_lo]

        def per_dpair(d2, _):
            col_a = jnp.full((16,), 2 * d2, jnp.int32)
            col_b = col_a + 1
            for lc in range(NFULL + 1):
                va = plsc.load_gather(rows_v, [l_idx[lc], col_a])
                vb = plsc.load_gather(rows_v, [l_idx[lc], col_b])
                lo = lax.shift_right_logical(va, sh_lo[lc]) & LOMASK
                hi = lax.shift_left(vb, sh_hi[lc]) & HIMASK
                word = lo | hi
                if lc < NFULL:
                    out_v[d2, pl.ds(lc * 16, 16)] = word
                else:
                    row = jnp.full((16,), d2, jnp.int32)
                    plsc.store_scatter(out_v, [row, l_idx[lc]], word,
                                       mask=tail_mask)
            return _

        lax.fori_loop(0, EMBED // 2, per_dpair, None)
        pltpu.sync_copy(out_v.bitcast(jnp.float16), out_hbm.at[b])
        return _

    lax.fori_loop(0, BPW, per_batch, None)


@jax.jit
def kernel(ids, weight):
    ids = ids.astype(jnp.int32)
    ids_half = (ids >> 1).reshape(BATCH, 2, HIST // 2)
    par16 = (ids & 1) << 4

    mesh = plsc.VectorSubcoreMesh(core_axis_name="c", subcore_axis_name="s")
    return pl.kernel(
        _body,
        out_type=jax.ShapeDtypeStruct((BATCH, EMBED, HIST), jnp.float16),
        mesh=mesh,
        scratch_types=[
            pltpu.VMEM((2, HIST // 2), jnp.int32),
            pltpu.VMEM((HIST,), jnp.int32),
            pltpu.VMEM((HIST, EMBED), jnp.int32),
            pltpu.VMEM((EMBED // 2, HIST), jnp.int32),
            pltpu.SemaphoreType.DMA,
        ],
        compiler_params=pltpu.CompilerParams(needs_layout_passes=False),
    )(ids_half, par16, weight)


# speedup vs baseline: 1.1821x; 1.1821x over previous
import jax
import jax.numpy as jnp
import numpy as np
from jax import lax
from jax.experimental import pallas as pl
from jax.experimental.pallas import tpu as pltpu
from jax.experimental.pallas import tpu_sc as plsc

VOCAB = 100000
EMBED = 128
BATCH = 4096
HIST = 200

NWORKERS = 32
BPW = BATCH // NWORKERS
LOMASK = np.int32(0xFFFF)
HIMASK = np.int32(-65536)
NCHUNK = (HIST + 15) // 16
NTAIL = HIST - (NCHUNK - 1) * 16


def _body(ids_hbm, weight_hbm, out_hbm, ids_all, idx_v, rows_v, out_v,
          gsem0, gsem1, osem):
    wid = lax.axis_index("s") * 2 + lax.axis_index("c")
    base_b = wid * BPW
    weight_i32 = weight_hbm.bitcast(jnp.int32)
    gsems = (gsem0, gsem1)

    iota = lax.iota(jnp.int32, 16)
    l_idx = [jnp.minimum(lc * 16 + iota, HIST - 1) for lc in range(NCHUNK)]
    l_row = [(l >= 100).astype(jnp.int32) for l in l_idx]
    l_col = [l - 100 * r for l, r in zip(l_idx, l_row)]
    tail_mask = iota < NTAIL
    c_tail = jnp.minimum(96 + iota, 99)

    pltpu.sync_copy(ids_hbm.at[pl.ds(base_b, BPW)], ids_all)

    def prepare(bn, buf):
        for r in range(2):
            for c in range(6):
                v = ids_all[bn, r, pl.ds(c * 16, 16)]
                idx_v[buf, r, pl.ds(c * 16, 16)] = (
                    lax.shift_right_logical(v, 1))
            v = plsc.load_gather(
                ids_all, [jnp.full((16,), bn, jnp.int32),
                          jnp.full((16,), r, jnp.int32), c_tail])
            idx_v[buf, r, pl.ds(96, 16)] = lax.shift_right_logical(v, 1)
        for r in range(2):
            pltpu.async_copy(
                weight_i32.at[idx_v.at[buf, r, pl.ds(0, 100)]],
                rows_v.at[buf, pl.ds(r * 100, 100)], gsems[buf])

    def transpose(bn, buf):
        bsp = jnp.full((16,), bn, jnp.int32)
        sh_lo = []
        for lc in range(NCHUNK):
            raw = plsc.load_gather(ids_all, [bsp, l_row[lc], l_col[lc]])
            sh_lo.append(lax.shift_left(raw & 1, 4))
        sh_hi = [16 - s for s in sh_lo]

        def per_dpair(d2, _):
            col_a = jnp.full((16,), 2 * d2, jnp.int32)
            col_b = col_a + 1
            for lc in range(NCHUNK):
                va = plsc.load_gather(rows_v.at[buf], [l_idx[lc], col_a])
                vb = plsc.load_gather(rows_v.at[buf], [l_idx[lc], col_b])
                lo = lax.shift_right_logical(va, sh_lo[lc]) & LOMASK
                hi = lax.shift_left(vb, sh_hi[lc]) & HIMASK
                word = lo | hi
                if lc < NCHUNK - 1:
                    out_v[buf, d2, pl.ds(lc * 16, 16)] = word
                else:
                    plsc.store_scatter(
                        out_v.at[buf],
                        [jnp.full((16,), d2, jnp.int32), l_idx[lc]],
                        word, mask=tail_mask)
            return _

        lax.fori_loop(0, EMBED // 2, per_dpair, None)

    def wait_gather(buf):
        pltpu.make_async_copy(
            weight_i32.at[pl.ds(0, HIST)], rows_v.at[buf],
            gsems[buf]).wait()

    def drain_out(buf):
        pltpu.make_async_copy(
            out_hbm.at[base_b], out_v.at[buf].bitcast(jnp.float16),
            osem).wait()

    prepare(0, 0)

    def step(k, _):
        for buf in range(2):
            bi = 2 * k + buf
            prepare(jnp.minimum(bi + 1, BPW - 1), 1 - buf)
            wait_gather(buf)

            @pl.when(k >= 1)
            def _drain():
                drain_out(buf)

            transpose(bi, buf)
            pltpu.async_copy(
                out_v.at[buf].bitcast(jnp.float16),
                out_hbm.at[base_b + bi], osem)
        return _

    lax.fori_loop(0, BPW // 2, step, None)

    wait_gather(0)
    drain_out(0)
    drain_out(1)


@jax.jit
def kernel(ids, weight):
    ids3 = ids.astype(jnp.int32).reshape(BATCH, 2, HIST // 2)

    mesh = plsc.VectorSubcoreMesh(core_axis_name="c", subcore_axis_name="s")
    return pl.kernel(
        _body,
        out_type=jax.ShapeDtypeStruct((BATCH, EMBED, HIST), jnp.float16),
        mesh=mesh,
        scratch_types=[
            pltpu.VMEM((BPW, 2, HIST // 2), jnp.int32),
            pltpu.VMEM((2, 2, 112), jnp.int32),
            pltpu.VMEM((2, HIST, EMBED), jnp.int32),
            pltpu.VMEM((2, EMBED // 2, HIST), jnp.int32),
            pltpu.SemaphoreType.DMA,
            pltpu.SemaphoreType.DMA,
            pltpu.SemaphoreType.DMA,
        ],
        compiler_params=pltpu.CompilerParams(needs_layout_passes=False),
    )(ids3, weight)


# speedup vs baseline: 1.5890x; 1.3443x over previous
import jax
import jax.numpy as jnp
import numpy as np
from jax import lax
from jax.experimental import pallas as pl
from jax.experimental.pallas import tpu as pltpu
from jax.experimental.pallas import tpu_sc as plsc

VOCAB = 100000
EMBED = 128
BATCH = 4096
HIST = 200

NWORKERS = 32
BPW = BATCH // NWORKERS
LOMASK = np.int32(0xFFFF)
HIMASK = np.int32(-65536)
NCHUNK = (HIST + 15) // 16
NTAIL = HIST - (NCHUNK - 1) * 16


def _body(ids_hbm, weight_hbm, out_hbm, ids_all, idx_v, rows_v, out_v,
          gsem0, gsem1, osem):
    wid = lax.axis_index("s") * 2 + lax.axis_index("c")
    base_b = wid * BPW
    weight_i32 = weight_hbm.bitcast(jnp.int32)
    gsems = (gsem0, gsem1)

    iota = lax.iota(jnp.int32, 16)
    l_idx = [jnp.minimum(lc * 16 + iota, HIST - 1) for lc in range(NCHUNK)]
    l_row = [(l >= 100).astype(jnp.int32) for l in l_idx]
    l_col = [l - 100 * r for l, r in zip(l_idx, l_row)]
    tail_mask = iota < NTAIL
    c_tail = jnp.minimum(96 + iota, 99)

    pltpu.sync_copy(ids_hbm.at[pl.ds(base_b, BPW)], ids_all)

    def prepare(bn, buf):
        for r in range(2):
            for c in range(6):
                v = ids_all[bn, r, pl.ds(c * 16, 16)]
                idx_v[buf, r, pl.ds(c * 16, 16)] = (
                    lax.shift_right_logical(v, 1))
            v = plsc.load_gather(
                ids_all, [jnp.full((16,), bn, jnp.int32),
                          jnp.full((16,), r, jnp.int32), c_tail])
            idx_v[buf, r, pl.ds(96, 16)] = lax.shift_right_logical(v, 1)
        for r in range(2):
            pltpu.async_copy(
                weight_i32.at[idx_v.at[buf, r, pl.ds(0, 100)]],
                rows_v.at[buf, pl.ds(r * 100, 100)], gsems[buf])

    def transpose(bn, buf):
        bsp = jnp.full((16,), bn, jnp.int32)
        sh_lo = []
        for lc in range(NCHUNK):
            raw = plsc.load_gather(ids_all, [bsp, l_row[lc], l_col[lc]])
            sh_lo.append(lax.shift_left(raw & 1, 4))
        sh_hi = [16 - s for s in sh_lo]

        @plsc.parallel_loop(0, EMBED // 2, unroll=4)
        def per_dpair(d2):
            col_a = jnp.full((16,), 2 * d2, jnp.int32)
            col_b = col_a + 1
            for lc in range(NCHUNK):
                va = plsc.load_gather(rows_v.at[buf], [l_idx[lc], col_a])
                vb = plsc.load_gather(rows_v.at[buf], [l_idx[lc], col_b])
                lo = lax.shift_right_logical(va, sh_lo[lc]) & LOMASK
                hi = lax.shift_left(vb, sh_hi[lc]) & HIMASK
                word = lo | hi
                if lc < NCHUNK - 1:
                    out_v[buf, d2, pl.ds(lc * 16, 16)] = word
                else:
                    plsc.store_scatter(
                        out_v.at[buf],
                        [jnp.full((16,), d2, jnp.int32), l_idx[lc]],
                        word, mask=tail_mask)

    def wait_gather(buf):
        pltpu.make_async_copy(
            weight_i32.at[pl.ds(0, HIST)], rows_v.at[buf],
            gsems[buf]).wait()

    def drain_out(buf):
        pltpu.make_async_copy(
            out_hbm.at[base_b], out_v.at[buf].bitcast(jnp.float16),
            osem).wait()

    prepare(0, 0)

    def step(k, _):
        for buf in range(2):
            bi = 2 * k + buf
            prepare(jnp.minimum(bi + 1, BPW - 1), 1 - buf)
            wait_gather(buf)

            @pl.when(k >= 1)
            def _drain():
                drain_out(buf)

            transpose(bi, buf)
            pltpu.async_copy(
                out_v.at[buf].bitcast(jnp.float16),
                out_hbm.at[base_b + bi], osem)
        return _

    lax.fori_loop(0, BPW // 2, step, None)

    wait_gather(0)
    drain_out(0)
    drain_out(1)


@jax.jit
def kernel(ids, weight):
    ids3 = ids.astype(jnp.int32).reshape(BATCH, 2, HIST // 2)

    mesh = plsc.VectorSubcoreMesh(core_axis_name="c", subcore_axis_name="s")
    return pl.kernel(
        _body,
        out_type=jax.ShapeDtypeStruct((BATCH, EMBED, HIST), jnp.float16),
        mesh=mesh,
        scratch_types=[
            pltpu.VMEM((BPW, 2, HIST // 2), jnp.int32),
            pltpu.VMEM((2, 2, 112), jnp.int32),
            pltpu.VMEM((2, HIST, EMBED), jnp.int32),
            pltpu.VMEM((2, EMBED // 2, HIST), jnp.int32),
            pltpu.SemaphoreType.DMA,
            pltpu.SemaphoreType.DMA,
            pltpu.SemaphoreType.DMA,
        ],
        compiler_params=pltpu.CompilerParams(needs_layout_passes=False),
    )(ids3, weight)


# speedup vs baseline: 1.9708x; 1.2403x over previous
import jax
import jax.numpy as jnp
import numpy as np
from jax import lax
from jax.experimental import pallas as pl
from jax.experimental.pallas import tpu as pltpu
from jax.experimental.pallas import tpu_sc as plsc

VOCAB = 100000
EMBED = 128
BATCH = 4096
HIST = 200

NWORKERS = 32
BPW = BATCH // NWORKERS
NCHUNK = (HIST + 15) // 16
NTAIL = HIST - (NCHUNK - 1) * 16


def _body(ids_hbm, weight_hbm, out_hbm, ids_all, rows_v, out_v,
          gsem0, gsem1, osem):
    wid = lax.axis_index("s") * 2 + lax.axis_index("c")
    base_b = wid * BPW
    gsems = (gsem0, gsem1)

    iota = lax.iota(jnp.int32, 16)
    l_idx = [jnp.minimum(lc * 16 + iota, HIST - 1) for lc in range(NCHUNK)]
    tail_mask = iota < NTAIL

    pltpu.sync_copy(ids_hbm.at[pl.ds(base_b, BPW)], ids_all)

    def prepare(bn, buf):
        for r in range(2):
            pltpu.async_copy(
                weight_hbm.at[ids_all.at[bn, r]],
                rows_v.at[buf, pl.ds(r * 100, 100)], gsems[buf])

    def transpose(buf):
        @plsc.parallel_loop(0, EMBED // 2, unroll=8)
        def per_dpair(d2):
            col = jnp.full((16,), d2, jnp.int32)
            for lc in range(NCHUNK):
                word = plsc.load_gather(rows_v.at[buf], [l_idx[lc], col])
                if lc < NCHUNK - 1:
                    out_v[buf, d2, pl.ds(lc * 16, 16)] = word
                else:
                    plsc.store_scatter(
                        out_v.at[buf],
                        [jnp.full((16,), d2, jnp.int32), l_idx[lc]],
                        word, mask=tail_mask)

    def wait_gather(buf):
        pltpu.make_async_copy(
            weight_hbm.at[pl.ds(0, HIST)], rows_v.at[buf],
            gsems[buf]).wait()

    def drain_out(buf):
        pltpu.make_async_copy(
            out_hbm.at[base_b], out_v.at[buf].bitcast(jnp.float16),
            osem).wait()

    prepare(0, 0)

    def step(k, _):
        for buf in range(2):
            bi = 2 * k + buf
            prepare(jnp.minimum(bi + 1, BPW - 1), 1 - buf)
            wait_gather(buf)

            @pl.when(k >= 1)
            def _drain():
                drain_out(buf)

            transpose(buf)
            pltpu.async_copy(
                out_v.at[buf].bitcast(jnp.float16),
                out_hbm.at[base_b + bi], osem)
        return _

    lax.fori_loop(0, BPW // 2, step, None)

    wait_gather(0)
    drain_out(0)
    drain_out(1)


@jax.jit
def kernel(ids, weight):
    ids3 = ids.astype(jnp.int32).reshape(BATCH, 2, HIST // 2)
    w64 = lax.bitcast_convert_type(
        weight.reshape(VOCAB, EMBED // 2, 2), jnp.int32)
    wpad = jnp.pad(w64, ((0, 0), (0, EMBED // 2)))

    mesh = plsc.VectorSubcoreMesh(core_axis_name="c", subcore_axis_name="s")
    return pl.kernel(
        _body,
        out_type=jax.ShapeDtypeStruct((BATCH, EMBED, HIST), jnp.float16),
        mesh=mesh,
        scratch_types=[
            pltpu.VMEM((BPW, 2, HIST // 2), jnp.int32),
            pltpu.VMEM((2, HIST, EMBED), jnp.int32),
            pltpu.VMEM((2, EMBED // 2, HIST), jnp.int32),
            pltpu.SemaphoreType.DMA,
            pltpu.SemaphoreType.DMA,
            pltpu.SemaphoreType.DMA,
        ],
        compiler_params=pltpu.CompilerParams(needs_layout_passes=False),
    )(ids3, wpad)
